# baseline (speedup 1.0000x reference)
import jax
import jax.numpy as jnp
from jax.experimental import pallas as pl
from jax.experimental.pallas import tpu as pltpu

N_EXPERTS = 16
INPUT_DIM = 1024
HIDDEN = 128
OUTPUT_DIM = 1024
B = 2048
K = 8
EH = N_EXPERTS * HIDDEN
BT = 256


def _moe_kernel(x_ref, gw_ref, gb_ref, w1_ref, b1t_ref, w2f_ref, b2m_ref,
                s_ref, o_ref):
    x = x_ref[...]
    lt = jax.lax.dot_general(
        gw_ref[...], x, (((1,), (1,)), ((), ())),
        preferred_element_type=jnp.float32) + gb_ref[...]
    cand = lt[:, None, :]
    comp = lt[None, :, :]
    icand = jax.lax.broadcasted_iota(
        jnp.int32, (N_EXPERTS, N_EXPERTS, 1), 0)
    icomp = jax.lax.broadcasted_iota(
        jnp.int32, (N_EXPERTS, N_EXPERTS, 1), 1)
    beats = (comp > cand) | ((comp == cand) & (icomp < icand))
    rank = jnp.sum(beats.astype(jnp.int32), axis=1)
    sel = rank < K
    masked = jnp.where(sel, lt, -jnp.inf)
    m = jnp.max(masked, axis=0, keepdims=True)
    p = jnp.where(sel, jnp.exp(lt - m), 0.0)
    wsel_t = p / jnp.sum(p, axis=0, keepdims=True)

    ht = jax.lax.dot_general(
        w1_ref[...], x, (((1,), (1,)), ((), ())),
        preferred_element_type=jnp.float32)
    ht = ht.reshape(EH, BT) + b1t_ref[...]
    scale_t = jax.lax.dot_general(
        s_ref[...], wsel_t, (((0,), (0,)), ((), ())),
        preferred_element_type=jnp.float32)
    gt = jnp.maximum(ht, 0.0) * scale_t
    out = jax.lax.dot_general(
        gt, w2f_ref[...], (((0,), (0,)), ((), ())),
        preferred_element_type=jnp.float32)
    out += jax.lax.dot_general(
        wsel_t, b2m_ref[...], (((0,), (0,)), ((), ())),
        preferred_element_type=jnp.float32)
    o_ref[...] = out


@jax.jit
def _moe(x, gate_W, gate_b, expert_w1, expert_b1, expert_w2, expert_b2):
    gb = gate_b.reshape(N_EXPERTS, 1)
    b1t = expert_b1.reshape(EH, 1)
    w2f = expert_w2.reshape(EH, OUTPUT_DIM)
    cols = jnp.arange(EH, dtype=jnp.int32) // HIDDEN
    s = (cols[None, :] == jnp.arange(N_EXPERTS, dtype=jnp.int32)[:, None]
         ).astype(jnp.float32)
    return pl.pallas_call(
        _moe_kernel,
        grid=(B // BT,),
        in_specs=[
            pl.BlockSpec((BT, INPUT_DIM), lambda i: (i, 0)),
            pl.BlockSpec((N_EXPERTS, INPUT_DIM), lambda i: (0, 0)),
            pl.BlockSpec((N_EXPERTS, 1), lambda i: (0, 0)),
            pl.BlockSpec((N_EXPERTS, INPUT_DIM, HIDDEN), lambda i: (0, 0, 0)),
            pl.BlockSpec((EH, 1), lambda i: (0, 0)),
            pl.BlockSpec((EH, OUTPUT_DIM), lambda i: (0, 0)),
            pl.BlockSpec((N_EXPERTS, OUTPUT_DIM), lambda i: (0, 0)),
            pl.BlockSpec((N_EXPERTS, EH), lambda i: (0, 0)),
        ],
        out_specs=pl.BlockSpec((BT, OUTPUT_DIM), lambda i: (i, 0)),
        out_shape=jax.ShapeDtypeStruct((B, OUTPUT_DIM), jnp.float32),
    )(x, gate_W, gb, expert_w1, b1t, w2f, expert_b2, s)


def kernel(x, gate_W, gate_b, expert_w1, expert_b1, expert_w2, expert_b2, k):
    del k
    return _moe(x, gate_W, gate_b, expert_w1, expert_b1, expert_w2, expert_b2)


# speedup vs baseline: 1.3690x; 1.3690x over previous
import jax
import jax.numpy as jnp
from jax.experimental import pallas as pl
from jax.experimental.pallas import tpu as pltpu

N_EXPERTS = 16
INPUT_DIM = 1024
HIDDEN = 128
OUTPUT_DIM = 1024
B = 2048
K = 8
EH = N_EXPERTS * HIDDEN
BT = 1024


def _moe_kernel(x_ref, gw_ref, gb_ref, w1_ref, b1t_ref, w2f_ref, b2m_ref,
                s_ref, o_ref):
    x = x_ref[...]
    lt = jax.lax.dot_general(
        gw_ref[...], x, (((1,), (1,)), ((), ())),
        preferred_element_type=jnp.float32) + gb_ref[...]
    cand = lt[:, None, :]
    comp = lt[None, :, :]
    icand = jax.lax.broadcasted_iota(
        jnp.int32, (N_EXPERTS, N_EXPERTS, 1), 0)
    icomp = jax.lax.broadcasted_iota(
        jnp.int32, (N_EXPERTS, N_EXPERTS, 1), 1)
    beats = (comp > cand) | ((comp == cand) & (icomp < icand))
    rank = jnp.sum(beats.astype(jnp.int32), axis=1)
    sel = rank < K
    masked = jnp.where(sel, lt, -jnp.inf)
    m = jnp.max(masked, axis=0, keepdims=True)
    p = jnp.where(sel, jnp.exp(lt - m), 0.0)
    wsel_t = p / jnp.sum(p, axis=0, keepdims=True)

    ht = jax.lax.dot_general(
        w1_ref[...], x, (((1,), (1,)), ((), ())),
        preferred_element_type=jnp.float32)
    ht = ht.reshape(EH, BT) + b1t_ref[...]
    scale_t = jax.lax.dot_general(
        s_ref[...], wsel_t, (((0,), (0,)), ((), ())),
        preferred_element_type=jnp.float32)
    gt = jnp.maximum(ht, 0.0) * scale_t
    out = jax.lax.dot_general(
        gt, w2f_ref[...], (((0,), (0,)), ((), ())),
        preferred_element_type=jnp.float32)
    out += jax.lax.dot_general(
        wsel_t, b2m_ref[...], (((0,), (0,)), ((), ())),
        preferred_element_type=jnp.float32)
    o_ref[...] = out


@jax.jit
def _moe(x, gate_W, gate_b, expert_w1, expert_b1, expert_w2, expert_b2):
    gb = gate_b.reshape(N_EXPERTS, 1)
    b1t = expert_b1.reshape(EH, 1)
    w2f = expert_w2.reshape(EH, OUTPUT_DIM)
    cols = jnp.arange(EH, dtype=jnp.int32) // HIDDEN
    s = (cols[None, :] == jnp.arange(N_EXPERTS, dtype=jnp.int32)[:, None]
         ).astype(jnp.float32)
    return pl.pallas_call(
        _moe_kernel,
        grid=(B // BT,),
        in_specs=[
            pl.BlockSpec((BT, INPUT_DIM), lambda i: (i, 0)),
            pl.BlockSpec((N_EXPERTS, INPUT_DIM), lambda i: (0, 0)),
            pl.BlockSpec((N_EXPERTS, 1), lambda i: (0, 0)),
            pl.BlockSpec((N_EXPERTS, INPUT_DIM, HIDDEN), lambda i: (0, 0, 0)),
            pl.BlockSpec((EH, 1), lambda i: (0, 0)),
            pl.BlockSpec((EH, OUTPUT_DIM), lambda i: (0, 0)),
            pl.BlockSpec((N_EXPERTS, OUTPUT_DIM), lambda i: (0, 0)),
            pl.BlockSpec((N_EXPERTS, EH), lambda i: (0, 0)),
        ],
        out_specs=pl.BlockSpec((BT, OUTPUT_DIM), lambda i: (i, 0)),
        out_shape=jax.ShapeDtypeStruct((B, OUTPUT_DIM), jnp.float32),
    )(x, gate_W, gb, expert_w1, b1t, w2f, expert_b2, s)


def kernel(x, gate_W, gate_b, expert_w1, expert_b1, expert_w2, expert_b2, k):
    del k
    return _moe(x, gate_W, gate_b, expert_w1, expert_b1, expert_w2, expert_b2)


# speedup vs baseline: 1.3891x; 1.0147x over previous
import jax
import jax.numpy as jnp
from jax.experimental import pallas as pl
from jax.experimental.pallas import tpu as pltpu

N_EXPERTS = 16
INPUT_DIM = 1024
HIDDEN = 128
OUTPUT_DIM = 1024
B = 2048
K = 8
EH = N_EXPERTS * HIDDEN
BT = 512


def _moe_kernel(x_ref, gw_ref, gb_ref, w1_ref, b1t_ref, w2f_ref, b2m_ref,
                s_ref, o_ref):
    x = x_ref[...]
    lt = jax.lax.dot_general(
        gw_ref[...], x, (((1,), (1,)), ((), ())),
        preferred_element_type=jnp.float32) + gb_ref[...]
    cand = lt[:, None, :]
    comp = lt[None, :, :]
    icand = jax.lax.broadcasted_iota(
        jnp.int32, (N_EXPERTS, N_EXPERTS, 1), 0)
    icomp = jax.lax.broadcasted_iota(
        jnp.int32, (N_EXPERTS, N_EXPERTS, 1), 1)
    beats = (comp > cand) | ((comp == cand) & (icomp < icand))
    rank = jnp.sum(beats.astype(jnp.int32), axis=1)
    sel = rank < K
    masked = jnp.where(sel, lt, -jnp.inf)
    m = jnp.max(masked, axis=0, keepdims=True)
    p = jnp.where(sel, jnp.exp(lt - m), 0.0)
    wsel_t = p / jnp.sum(p, axis=0, keepdims=True)

    ht = jax.lax.dot_general(
        w1_ref[...], x, (((1,), (1,)), ((), ())),
        preferred_element_type=jnp.float32)
    ht = ht.reshape(EH, BT) + b1t_ref[...]
    scale_t = jax.lax.dot_general(
        s_ref[...], wsel_t, (((0,), (0,)), ((), ())),
        preferred_element_type=jnp.float32)
    gt = jnp.maximum(ht, 0.0) * scale_t
    out = jax.lax.dot_general(
        gt, w2f_ref[...], (((0,), (0,)), ((), ())),
        preferred_element_type=jnp.float32)
    out += jax.lax.dot_general(
        wsel_t, b2m_ref[...], (((0,), (0,)), ((), ())),
        preferred_element_type=jnp.float32)
    o_ref[...] = out


@jax.jit
def _moe(x, gate_W, gate_b, expert_w1, expert_b1, expert_w2, expert_b2):
    gb = gate_b.reshape(N_EXPERTS, 1)
    b1t = expert_b1.reshape(EH, 1)
    w2f = expert_w2.reshape(EH, OUTPUT_DIM)
    cols = jnp.arange(EH, dtype=jnp.int32) // HIDDEN
    s = (cols[None, :] == jnp.arange(N_EXPERTS, dtype=jnp.int32)[:, None]
         ).astype(jnp.float32)
    return pl.pallas_call(
        _moe_kernel,
        grid=(B // BT,),
        in_specs=[
            pl.BlockSpec((BT, INPUT_DIM), lambda i: (i, 0)),
            pl.BlockSpec((N_EXPERTS, INPUT_DIM), lambda i: (0, 0)),
            pl.BlockSpec((N_EXPERTS, 1), lambda i: (0, 0)),
            pl.BlockSpec((N_EXPERTS, INPUT_DIM, HIDDEN), lambda i: (0, 0, 0)),
            pl.BlockSpec((EH, 1), lambda i: (0, 0)),
            pl.BlockSpec((EH, OUTPUT_DIM), lambda i: (0, 0)),
            pl.BlockSpec((N_EXPERTS, OUTPUT_DIM), lambda i: (0, 0)),
            pl.BlockSpec((N_EXPERTS, EH), lambda i: (0, 0)),
        ],
        out_specs=pl.BlockSpec((BT, OUTPUT_DIM), lambda i: (i, 0)),
        out_shape=jax.ShapeDtypeStruct((B, OUTPUT_DIM), jnp.float32),
    )(x, gate_W, gb, expert_w1, b1t, w2f, expert_b2, s)


def kernel(x, gate_W, gate_b, expert_w1, expert_b1, expert_w2, expert_b2, k):
    del k
    return _moe(x, gate_W, gate_b, expert_w1, expert_b1, expert_w2, expert_b2)
